# baseline (speedup 1.0000x reference)
import functools

import jax
import jax.numpy as jnp
from jax import lax
from jax.experimental import pallas as pl
from jax.experimental.pallas import tpu as pltpu
from jax.experimental.pallas import tpu_sc as plsc

NTOKEN = 1000000
EMB_DIM = 32
BATCH = 16384
HIST = 50

B = BATCH * HIST
NC, NS = 2, 16
NW = NC * NS
BPW = B // NW
CH = 1280
NCHUNK = BPW // CH

_mesh = plsc.VectorSubcoreMesh(core_axis_name="c", subcore_axis_name="s")


@functools.partial(
    pl.kernel,
    mesh=_mesh,
    out_type=jax.ShapeDtypeStruct((B, EMB_DIM), jnp.float32),
    scratch_types=[
        pltpu.VMEM((BPW,), jnp.int32),
        pltpu.VMEM((2, CH, EMB_DIM), jnp.float32),
        pltpu.SemaphoreType.DMA,
        pltpu.SemaphoreType.DMA,
        pltpu.SemaphoreType.DMA,
        pltpu.SemaphoreType.DMA,
    ],
    compiler_params=pltpu.CompilerParams(use_tc_tiling_on_sc=False),
)
def _gather_kernel(idx_hbm, table_hbm, out_hbm, idx_v, rows_v,
                   gsem0, gsem1, ssem0, ssem1):
    wid = lax.axis_index("s") * NC + lax.axis_index("c")
    base = wid * BPW
    pltpu.sync_copy(idx_hbm.at[pl.ds(base, BPW)], idx_v)

    def fire_gather(c, buf_ref, sem):
        pltpu.async_copy(
            table_hbm.at[idx_v.at[pl.ds(c * CH, CH)]], buf_ref, sem)

    def wait_gather(c, buf_ref, sem):
        pltpu.make_async_copy(
            table_hbm.at[idx_v.at[pl.ds(c * CH, CH)]], buf_ref, sem).wait()

    def fire_store(c, buf_ref, sem):
        pltpu.async_copy(
            buf_ref, out_hbm.at[pl.ds(base + c * CH, CH)], sem)

    def wait_store(c, buf_ref, sem):
        pltpu.make_async_copy(
            buf_ref, out_hbm.at[pl.ds(base + c * CH, CH)], sem).wait()

    buf0, buf1 = rows_v.at[0], rows_v.at[1]
    fire_gather(0, buf0, gsem0)

    def body(s, carry):
        c0 = 2 * s

        @pl.when(s > 0)
        def _():
            wait_store(c0 - 1, buf1, ssem1)

        fire_gather(c0 + 1, buf1, gsem1)
        wait_gather(c0, buf0, gsem0)
        fire_store(c0, buf0, ssem0)

        @pl.when(c0 + 2 < NCHUNK)
        def _():
            wait_store(c0, buf0, ssem0)
            fire_gather(c0 + 2, buf0, gsem0)

        wait_gather(c0 + 1, buf1, gsem1)
        fire_store(c0 + 1, buf1, ssem1)
        return carry

    lax.fori_loop(0, NCHUNK // 2, body, 0)
    wait_store(NCHUNK - 2, buf0, ssem0)
    wait_store(NCHUNK - 1, buf1, ssem1)


def kernel(x, table):
    idx = x.reshape(B).astype(jnp.int32)
    out = _gather_kernel(idx, table)
    return out.reshape(BATCH, HIST, EMB_DIM)


# speedup vs baseline: 1.1775x; 1.1775x over previous
import functools

import jax
import jax.numpy as jnp
from jax import lax
from jax.experimental import pallas as pl
from jax.experimental.pallas import tpu as pltpu
from jax.experimental.pallas import tpu_sc as plsc

NTOKEN = 1000000
EMB_DIM = 32
BATCH = 16384
HIST = 50

B = BATCH * HIST
NC, NS = 2, 16
NW = NC * NS
BPW = B // NW
ROW = 128
NROWS = BPW // ROW
K = 10
NCHUNK = NROWS // K

_mesh = plsc.VectorSubcoreMesh(core_axis_name="c", subcore_axis_name="s")


@functools.partial(
    pl.kernel,
    mesh=_mesh,
    out_type=jax.ShapeDtypeStruct((B // ROW, ROW, EMB_DIM), jnp.float32),
    scratch_types=[
        pltpu.VMEM((NROWS, ROW), jnp.int32),
        pltpu.VMEM((2, K, ROW, EMB_DIM), jnp.float32),
        pltpu.SemaphoreType.DMA,
        pltpu.SemaphoreType.DMA,
        pltpu.SemaphoreType.DMA,
        pltpu.SemaphoreType.DMA,
    ],
    compiler_params=pltpu.CompilerParams(use_tc_tiling_on_sc=False),
)
def _gather_kernel(idx_hbm, table_hbm, out_hbm, idx_v, rows_v,
                   gsem0, gsem1, ssem0, ssem1):
    wid = lax.axis_index("s") * NC + lax.axis_index("c")
    base = wid * NROWS
    pltpu.sync_copy(idx_hbm.at[pl.ds(base, NROWS)], idx_v)

    def fire_gathers(c, buf_ref, sem):
        for j in range(K):
            pltpu.async_copy(
                table_hbm.at[idx_v.at[c * K + j]], buf_ref.at[j], sem)

    def drain_gathers(c, buf_ref, sem):
        for j in range(K):
            pltpu.make_async_copy(
                table_hbm.at[idx_v.at[c * K + j]], buf_ref.at[j], sem).wait()

    def fire_store(c, buf_ref, sem):
        pltpu.async_copy(
            buf_ref, out_hbm.at[pl.ds(base + c * K, K)], sem)

    def wait_store(c, buf_ref, sem):
        pltpu.make_async_copy(
            buf_ref, out_hbm.at[pl.ds(base + c * K, K)], sem).wait()

    buf0, buf1 = rows_v.at[0], rows_v.at[1]
    fire_gathers(0, buf0, gsem0)
    fire_gathers(1, buf1, gsem1)

    def body(s, carry):
        c0 = 2 * s

        drain_gathers(c0, buf0, gsem0)
        fire_store(c0, buf0, ssem0)

        @pl.when(c0 + 2 < NCHUNK)
        def _():
            wait_store(c0, buf0, ssem0)
            fire_gathers(c0 + 2, buf0, gsem0)

        drain_gathers(c0 + 1, buf1, gsem1)
        fire_store(c0 + 1, buf1, ssem1)

        @pl.when(c0 + 3 < NCHUNK)
        def _():
            wait_store(c0 + 1, buf1, ssem1)
            fire_gathers(c0 + 3, buf1, gsem1)
        return carry

    lax.fori_loop(0, NCHUNK // 2, body, 0)
    wait_store(NCHUNK - 2, buf0, ssem0)
    wait_store(NCHUNK - 1, buf1, ssem1)


def kernel(x, table):
    idx = x.reshape(B // ROW, ROW).astype(jnp.int32)
    out = _gather_kernel(idx, table)
    return out.reshape(BATCH, HIST, EMB_DIM)


# speedup vs baseline: 1.4223x; 1.2079x over previous
import functools

import jax
import jax.numpy as jnp
from jax import lax
from jax.experimental import pallas as pl
from jax.experimental.pallas import tpu as pltpu
from jax.experimental.pallas import tpu_sc as plsc

NTOKEN = 1000000
EMB_DIM = 32
BATCH = 16384
HIST = 50

B = BATCH * HIST
NC, NS = 2, 16
NW = NC * NS
BPW = B // NW
ROW = 128
CB = 64
LOOK = CB * HIST
NST = LOOK // ROW
NCHUNK = BPW // LOOK
HG = 5
NHG = HIST // HG

_mesh = plsc.VectorSubcoreMesh(core_axis_name="c", subcore_axis_name="s")


@functools.partial(
    pl.kernel,
    mesh=_mesh,
    out_type=jax.ShapeDtypeStruct(
        (HIST, EMB_DIM // 8, BATCH // 128, 8, 128), jnp.float32),
    scratch_types=[
        pltpu.VMEM((2, NST, ROW), jnp.int32),
        pltpu.VMEM((LOOK, EMB_DIM), jnp.float32),
        pltpu.VMEM((2, HG, EMB_DIM // 8, 8, CB), jnp.float32),
        pltpu.SemaphoreType.DMA,
        pltpu.SemaphoreType.DMA,
        pltpu.SemaphoreType.DMA,
        pltpu.SemaphoreType.DMA,
    ],
    compiler_params=pltpu.CompilerParams(
        use_tc_tiling_on_sc=False, needs_layout_passes=False),
)
def _gather_kernel(idx_hbm, table_hbm, out_hbm, idx_v, gbuf, tbuf,
                   isem, gsem, ssem0, ssem1):
    wid = lax.axis_index("s") * NC + lax.axis_index("c")
    rbase = wid * (BPW // ROW)
    iota16 = lax.iota(jnp.int32, 16)
    rowmul = iota16 * HIST

    def fire_idx(c, buf):
        pltpu.async_copy(
            idx_hbm.at[pl.ds(rbase + c * NST, NST)], idx_v.at[buf], isem)

    def wait_idx(c, buf):
        pltpu.make_async_copy(
            idx_hbm.at[pl.ds(rbase + c * NST, NST)], idx_v.at[buf], isem
        ).wait()

    def store_dst(g, tc, l0):
        return out_hbm.at[pl.ds(g * HG, HG), :, tc, :, pl.ds(l0, CB)]

    ssems = (ssem0, ssem1)
    fire_idx(0, 0)

    def chunk(c, carry):
        cb = c % 2
        wait_idx(c, cb)
        for j in range(NST):
            pltpu.async_copy(
                table_hbm.at[idx_v.at[cb, j]],
                gbuf.at[pl.ds(j * ROW, ROW)], gsem)

        @pl.when(c + 1 < NCHUNK)
        def _():
            fire_idx(c + 1, 1 - cb)

        for j in range(NST):
            pltpu.make_async_copy(
                table_hbm.at[idx_v.at[cb, j]],
                gbuf.at[pl.ds(j * ROW, ROW)], gsem).wait()

        tc = wid * (BPW // (HIST * 128)) + c // 2
        l0 = (c % 2) * CB
        for g in range(NHG):
            tb = g % 2
            if g >= 2:
                pltpu.make_async_copy(
                    tbuf.at[tb], store_dst(g - 2, tc, l0), ssems[tb]).wait()
            else:
                @pl.when(c > 0)
                def _():
                    pltpu.make_async_copy(
                        tbuf.at[tb], store_dst(0, 0, 0), ssems[tb]).wait()

            g5 = g * HG

            @pl.loop(0, HG * EMB_DIM * (CB // 16), unroll=8)
            def _(i):
                q = i % 4
                d = (i // 4) % EMB_DIM
                hl = i // (4 * EMB_DIM)
                rows = rowmul + (q * (16 * HIST) + g5 + hl)
                cols = jnp.full((16,), d, jnp.int32)
                v = plsc.load_gather(gbuf, [rows, cols])
                tbuf[tb, hl, d // 8, d % 8, pl.ds(q * 16, 16)] = v

            pltpu.async_copy(tbuf.at[tb], store_dst(g, tc, l0), ssems[tb])
        return carry

    lax.fori_loop(0, NCHUNK, chunk, 0)
    pltpu.make_async_copy(tbuf.at[0], store_dst(0, 0, 0), ssem0).wait()
    pltpu.make_async_copy(tbuf.at[1], store_dst(0, 0, 0), ssem1).wait()


def kernel(x, table):
    idx = x.reshape(B // ROW, ROW).astype(jnp.int32)
    out5 = _gather_kernel(idx, table)
    return out5.transpose(2, 4, 0, 1, 3).reshape(BATCH, HIST, EMB_DIM)


# speedup vs baseline: 1.7170x; 1.2072x over previous
import functools

import jax
import jax.numpy as jnp
from jax import lax
from jax.experimental import pallas as pl
from jax.experimental.pallas import tpu as pltpu
from jax.experimental.pallas import tpu_sc as plsc

NTOKEN = 1000000
EMB_DIM = 32
BATCH = 16384
HIST = 50

B = BATCH * HIST
NC, NS = 2, 16
NW = NC * NS
BPW = B // NW
ROW = 128
CB = 64
LOOK = CB * HIST
NST = LOOK // ROW
NCHUNK = BPW // LOOK
HG = 5
NHG = HIST // HG

_mesh = plsc.VectorSubcoreMesh(core_axis_name="c", subcore_axis_name="s")


@functools.partial(
    pl.kernel,
    mesh=_mesh,
    out_type=jax.ShapeDtypeStruct(
        (HIST, EMB_DIM // 8, BATCH // 128, 8, 128), jnp.float32),
    scratch_types=[
        pltpu.VMEM((2, NST, ROW), jnp.int32),
        pltpu.VMEM((LOOK, EMB_DIM), jnp.float32),
        pltpu.VMEM((2, HG, EMB_DIM // 8, 8, CB), jnp.float32),
        pltpu.SemaphoreType.DMA,
        pltpu.SemaphoreType.DMA,
        pltpu.SemaphoreType.DMA,
        pltpu.SemaphoreType.DMA,
    ],
    compiler_params=pltpu.CompilerParams(
        use_tc_tiling_on_sc=False, needs_layout_passes=False),
)
def _gather_kernel(idx_hbm, table_hbm, out_hbm, idx_v, gbuf, tbuf,
                   isem, gsem, ssem0, ssem1):
    wid = lax.axis_index("s") * NC + lax.axis_index("c")
    rbase = wid * (BPW // ROW)
    iota16 = lax.iota(jnp.int32, 16)
    rowmul = iota16 * HIST

    def fire_idx(c, buf):
        pltpu.async_copy(
            idx_hbm.at[pl.ds(rbase + c * NST, NST)], idx_v.at[buf], isem)

    def wait_idx(c, buf):
        pltpu.make_async_copy(
            idx_hbm.at[pl.ds(rbase + c * NST, NST)], idx_v.at[buf], isem
        ).wait()

    def store_dst(g, tc, l0):
        return out_hbm.at[pl.ds(g * HG, HG), :, tc, :, pl.ds(l0, CB)]

    ssems = (ssem0, ssem1)
    fire_idx(0, 0)

    def chunk(c, carry):
        cb = c % 2
        wait_idx(c, cb)
        for j in range(NST):
            pltpu.async_copy(
                table_hbm.at[idx_v.at[cb, j]],
                gbuf.at[pl.ds(j * ROW, ROW)], gsem)

        @pl.when(c + 1 < NCHUNK)
        def _():
            fire_idx(c + 1, 1 - cb)

        for j in range(NST):
            pltpu.make_async_copy(
                table_hbm.at[idx_v.at[cb, j]],
                gbuf.at[pl.ds(j * ROW, ROW)], gsem).wait()

        tc = wid * (BPW // (HIST * 128)) + c // 2
        l0 = (c % 2) * CB
        for g in range(NHG):
            tb = g % 2
            if g >= 2:
                pltpu.make_async_copy(
                    tbuf.at[tb], store_dst(g - 2, tc, l0), ssems[tb]).wait()
            else:
                @pl.when(c > 0)
                def _():
                    pltpu.make_async_copy(
                        tbuf.at[tb], store_dst(0, 0, 0), ssems[tb]).wait()

            g5 = g * HG

            @plsc.parallel_loop(0, HG * EMB_DIM * (CB // 16), unroll=8)
            def _(i):
                q = i % 4
                d = (i // 4) % EMB_DIM
                hl = i // (4 * EMB_DIM)
                rows = rowmul + (q * (16 * HIST) + g5 + hl)
                cols = jnp.full((16,), d, jnp.int32)
                v = plsc.load_gather(gbuf, [rows, cols])
                tbuf[tb, hl, d // 8, d % 8, pl.ds(q * 16, 16)] = v

            pltpu.async_copy(tbuf.at[tb], store_dst(g, tc, l0), ssems[tb])
        return carry

    lax.fori_loop(0, NCHUNK, chunk, 0)
    pltpu.make_async_copy(tbuf.at[0], store_dst(0, 0, 0), ssem0).wait()
    pltpu.make_async_copy(tbuf.at[1], store_dst(0, 0, 0), ssem1).wait()


def kernel(x, table):
    idx = x.reshape(B // ROW, ROW).astype(jnp.int32)
    out5 = _gather_kernel(idx, table)
    return out5.transpose(2, 4, 0, 1, 3).reshape(BATCH, HIST, EMB_DIM)


# speedup vs baseline: 2.6789x; 1.5602x over previous
import functools

import jax
import jax.numpy as jnp
from jax import lax
from jax.experimental import pallas as pl
from jax.experimental.pallas import tpu as pltpu
from jax.experimental.pallas import tpu_sc as plsc

NTOKEN = 1000000
EMB_DIM = 32
BATCH = 16384
HIST = 50

B = BATCH * HIST
NC, NS = 2, 16
NW = NC * NS
BPW = B // NW
ROW = 128
CB = 64
LOOK = CB * HIST
NST = LOOK // ROW
NCHUNK = BPW // LOOK
HG = 5
NHG = HIST // HG
LP = CB + 1

_mesh = plsc.VectorSubcoreMesh(core_axis_name="c", subcore_axis_name="s")


@functools.partial(
    pl.kernel,
    mesh=_mesh,
    out_type=jax.ShapeDtypeStruct(
        (HIST, EMB_DIM // 8, BATCH // 128, 8, 128), jnp.float32),
    scratch_types=[
        pltpu.VMEM((NST, ROW), jnp.int32),
        pltpu.VMEM((LOOK, EMB_DIM), jnp.float32),
        pltpu.VMEM((2, HG, EMB_DIM // 8, 8, LP), jnp.float32),
        pltpu.SemaphoreType.DMA,
        pltpu.SemaphoreType.DMA,
        pltpu.SemaphoreType.DMA,
        pltpu.SemaphoreType.DMA,
    ],
    compiler_params=pltpu.CompilerParams(
        use_tc_tiling_on_sc=False, needs_layout_passes=False),
)
def _gather_kernel(idx_hbm, table_hbm, out_hbm, idx_v, gbuf, tbuf,
                   isem, gsem, ssem0, ssem1):
    wid = lax.axis_index("s") * NC + lax.axis_index("c")
    rbase = wid * (BPW // ROW)
    iota16 = lax.iota(jnp.int32, 16)
    s_vec = lax.rem(iota16, 8)
    tr0 = lax.div(iota16, 8)
    tr1 = tr0 + 2

    def load_idx(c):
        pltpu.sync_copy(idx_hbm.at[pl.ds(rbase + c * NST, NST)], idx_v)

    def store_src(tb):
        return tbuf.at[tb, :, :, :, pl.ds(0, CB)]

    def store_dst(g, tc, l0):
        return out_hbm.at[pl.ds(g * HG, HG), :, tc, :, pl.ds(l0, CB)]

    ssems = (ssem0, ssem1)

    def chunk(c, carry):
        load_idx(c)
        for j in range(NST):
            pltpu.async_copy(
                table_hbm.at[idx_v.at[j]],
                gbuf.at[pl.ds(j * ROW, ROW)], gsem)
        for j in range(NST):
            pltpu.make_async_copy(
                table_hbm.at[idx_v.at[j]],
                gbuf.at[pl.ds(j * ROW, ROW)], gsem).wait()

        tc = wid * (BPW // (HIST * 128)) + c // 2
        l0 = (c % 2) * CB
        for g in range(NHG):
            tb = g % 2
            if g >= 2:
                pltpu.make_async_copy(
                    store_src(tb), store_dst(g - 2, tc, l0), ssems[tb]).wait()
            else:
                @pl.when(c > 0)
                def _():
                    pltpu.make_async_copy(
                        store_src(tb), store_dst(0, 0, 0), ssems[tb]).wait()

            g5 = g * HG
            tbv = jnp.full((16,), tb, jnp.int32)

            @plsc.parallel_loop(0, HG * CB, unroll=8)
            def _(i):
                hl = lax.div(i, CB)
                bl = lax.rem(i, CB)
                p = bl * HIST + (g5 + hl)
                v0 = gbuf[p, pl.ds(0, 16)]
                v1 = gbuf[p, pl.ds(16, 16)]
                hlv = jnp.full((16,), hl, jnp.int32)
                blv = jnp.full((16,), bl, jnp.int32)
                plsc.store_scatter(tbuf, [tbv, hlv, tr0, s_vec, blv], v0)
                plsc.store_scatter(tbuf, [tbv, hlv, tr1, s_vec, blv], v1)

            pltpu.async_copy(store_src(tb), store_dst(g, tc, l0), ssems[tb])
        return carry

    lax.fori_loop(0, NCHUNK, chunk, 0)
    pltpu.make_async_copy(store_src(0), store_dst(0, 0, 0), ssem0).wait()
    pltpu.make_async_copy(store_src(1), store_dst(0, 0, 0), ssem1).wait()


def kernel(x, table):
    idx = x.reshape(B // ROW, ROW).astype(jnp.int32)
    out5 = _gather_kernel(idx, table)
    return out5.transpose(2, 4, 0, 1, 3).reshape(BATCH, HIST, EMB_DIM)


# speedup vs baseline: 2.6917x; 1.0048x over previous
import functools

import jax
import jax.numpy as jnp
from jax import lax
from jax.experimental import pallas as pl
from jax.experimental.pallas import tpu as pltpu
from jax.experimental.pallas import tpu_sc as plsc

NTOKEN = 1000000
EMB_DIM = 32
BATCH = 16384
HIST = 50

B = BATCH * HIST
NC, NS = 2, 16
NW = NC * NS
BPB = BATCH // NW
CB = 64
NCHUNK = BPB // CB
HG = 5
NHG = HIST // HG
LP = CB + 1

_mesh = plsc.VectorSubcoreMesh(core_axis_name="c", subcore_axis_name="s")


@functools.partial(
    pl.kernel,
    mesh=_mesh,
    out_type=jax.ShapeDtypeStruct(
        (HIST, EMB_DIM // 8, BATCH // 128, 8, 128), jnp.float32),
    scratch_types=[
        pltpu.VMEM((HIST, CB), jnp.int32),
        pltpu.VMEM((HIST, CB, EMB_DIM), jnp.float32),
        pltpu.VMEM((2, HG, EMB_DIM // 8, 8, LP), jnp.float32),
        pltpu.SemaphoreType.DMA,
        pltpu.SemaphoreType.DMA,
        pltpu.SemaphoreType.DMA,
        pltpu.SemaphoreType.DMA,
    ],
    compiler_params=pltpu.CompilerParams(
        use_tc_tiling_on_sc=False, needs_layout_passes=False),
)
def _gather_kernel(xt_hbm, table_hbm, out_hbm, idx_v, gbuf, tbuf,
                   isem, gsem, ssem0, ssem1):
    wid = lax.axis_index("s") * NC + lax.axis_index("c")
    b0 = wid * BPB
    iota16 = lax.iota(jnp.int32, 16)
    s_vec = lax.rem(iota16, 8)
    tr0 = lax.div(iota16, 8)
    tr1 = tr0 + 2

    def store_src(tb):
        return tbuf.at[tb, :, :, :, pl.ds(0, CB)]

    def store_dst(g, tc, l0):
        return out_hbm.at[pl.ds(g * HG, HG), :, tc, :, pl.ds(l0, CB)]

    ssems = (ssem0, ssem1)

    def chunk(c, carry):
        babs = b0 + c * CB
        pltpu.sync_copy(xt_hbm.at[:, pl.ds(babs, CB)], idx_v)

        @pl.loop(0, HIST)
        def _(j):
            pltpu.async_copy(
                table_hbm.at[idx_v.at[j]], gbuf.at[j], gsem)

        @pl.loop(0, HIST)
        def _(j):
            pltpu.make_async_copy(
                table_hbm.at[idx_v.at[j]], gbuf.at[j], gsem).wait()

        tc = wid * (BPB // 128) + c // 2
        l0 = (c % 2) * CB
        for g in range(NHG):
            tb = g % 2
            if g >= 2:
                pltpu.make_async_copy(
                    store_src(tb), store_dst(g - 2, tc, l0), ssems[tb]).wait()
            else:
                @pl.when(c > 0)
                def _():
                    pltpu.make_async_copy(
                        store_src(tb), store_dst(0, 0, 0), ssems[tb]).wait()

            g5 = g * HG
            tbv = jnp.full((16,), tb, jnp.int32)

            @plsc.parallel_loop(0, HG * CB, unroll=8)
            def _(i):
                hl = lax.div(i, CB)
                bl = lax.rem(i, CB)
                h = g5 + hl
                v0 = gbuf[h, bl, pl.ds(0, 16)]
                v1 = gbuf[h, bl, pl.ds(16, 16)]
                hlv = jnp.full((16,), hl, jnp.int32)
                blv = jnp.full((16,), bl, jnp.int32)
                plsc.store_scatter(tbuf, [tbv, hlv, tr0, s_vec, blv], v0)
                plsc.store_scatter(tbuf, [tbv, hlv, tr1, s_vec, blv], v1)

            pltpu.async_copy(store_src(tb), store_dst(g, tc, l0), ssems[tb])
        return carry

    lax.fori_loop(0, NCHUNK, chunk, 0)
    pltpu.make_async_copy(store_src(0), store_dst(0, 0, 0), ssem0).wait()
    pltpu.make_async_copy(store_src(1), store_dst(0, 0, 0), ssem1).wait()


def kernel(x, table):
    xt = x.T.astype(jnp.int32)
    out5 = _gather_kernel(xt, table)
    return out5.transpose(2, 4, 0, 1, 3).reshape(BATCH, HIST, EMB_DIM)


# speedup vs baseline: 2.7138x; 1.0082x over previous
import functools

import jax
import jax.numpy as jnp
from jax import lax
from jax.experimental import pallas as pl
from jax.experimental.pallas import tpu as pltpu
from jax.experimental.pallas import tpu_sc as plsc

NTOKEN = 1000000
EMB_DIM = 32
BATCH = 16384
HIST = 50

B = BATCH * HIST
NC, NS = 2, 16
NW = NC * NS
BPB = BATCH // NW
CB = 64
NCHUNK = BPB // CB
HG = 2
NHG = HIST // HG
LP = CB + 1

_mesh = plsc.VectorSubcoreMesh(core_axis_name="c", subcore_axis_name="s")


@functools.partial(
    pl.kernel,
    mesh=_mesh,
    out_type=jax.ShapeDtypeStruct(
        (HIST, EMB_DIM // 8, BATCH // 128, 8, 128), jnp.float32),
    scratch_types=[
        pltpu.VMEM((2, HIST, CB), jnp.int32),
        pltpu.VMEM((HIST, CB, EMB_DIM), jnp.float32),
        pltpu.VMEM((2, HG, EMB_DIM // 8, 8, LP), jnp.float32),
        pltpu.SemaphoreType.DMA,
        pltpu.SemaphoreType.DMA,
        pltpu.SemaphoreType.DMA,
        pltpu.SemaphoreType.DMA,
    ],
    compiler_params=pltpu.CompilerParams(
        use_tc_tiling_on_sc=False, needs_layout_passes=False),
)
def _gather_kernel(xt_hbm, table_hbm, out_hbm, idx_v, gbuf, tbuf,
                   isem, gsem, ssem0, ssem1):
    wid = lax.axis_index("s") * NC + lax.axis_index("c")
    b0 = wid * BPB
    iota16 = lax.iota(jnp.int32, 16)
    s_vec = lax.rem(iota16, 8)
    tr0 = lax.div(iota16, 8)
    tr1 = tr0 + 2

    def store_src(tb):
        return tbuf.at[tb, :, :, :, pl.ds(0, CB)]

    def store_dst(g, tc, l0):
        return out_hbm.at[pl.ds(g * HG, HG), :, tc, :, pl.ds(l0, CB)]

    ssems = (ssem0, ssem1)

    def fire_idx(c, ib):
        pltpu.async_copy(
            xt_hbm.at[:, pl.ds(b0 + c * CB, CB)], idx_v.at[ib], isem)

    fire_idx(0, 0)

    def chunk(c, carry):
        ib = c % 2
        pltpu.make_async_copy(
            xt_hbm.at[:, pl.ds(b0, CB)], idx_v.at[ib], isem).wait()

        @pl.loop(0, HIST)
        def _(j):
            pltpu.async_copy(
                table_hbm.at[idx_v.at[ib, j]], gbuf.at[j], gsem)

        @pl.when(c + 1 < NCHUNK)
        def _():
            fire_idx(c + 1, 1 - ib)

        @pl.loop(0, HIST)
        def _(j):
            pltpu.make_async_copy(
                table_hbm.at[idx_v.at[ib, j]], gbuf.at[j], gsem).wait()

        tc = wid * (BPB // 128) + c // 2
        l0 = (c % 2) * CB
        for g in range(NHG):
            tb = g % 2
            if g >= 2:
                pltpu.make_async_copy(
                    store_src(tb), store_dst(g - 2, tc, l0), ssems[tb]).wait()
            else:
                @pl.when(c > 0)
                def _():
                    pltpu.make_async_copy(
                        store_src(tb), store_dst(0, 0, 0), ssems[tb]).wait()

            g5 = g * HG
            tbv = jnp.full((16,), tb, jnp.int32)

            @plsc.parallel_loop(0, HG * CB, unroll=8)
            def _(i):
                hl = lax.div(i, CB)
                bl = lax.rem(i, CB)
                h = g5 + hl
                v0 = gbuf[h, bl, pl.ds(0, 16)]
                v1 = gbuf[h, bl, pl.ds(16, 16)]
                hlv = jnp.full((16,), hl, jnp.int32)
                blv = jnp.full((16,), bl, jnp.int32)
                plsc.store_scatter(tbuf, [tbv, hlv, tr0, s_vec, blv], v0)
                plsc.store_scatter(tbuf, [tbv, hlv, tr1, s_vec, blv], v1)

            pltpu.async_copy(store_src(tb), store_dst(g, tc, l0), ssems[tb])
        return carry

    lax.fori_loop(0, NCHUNK, chunk, 0)
    pltpu.make_async_copy(store_src(0), store_dst(0, 0, 0), ssem0).wait()
    pltpu.make_async_copy(store_src(1), store_dst(0, 0, 0), ssem1).wait()


def kernel(x, table):
    xt = x.T.astype(jnp.int32)
    out5 = _gather_kernel(xt, table)
    return out5.transpose(2, 4, 0, 1, 3).reshape(BATCH, HIST, EMB_DIM)
